# baseline (speedup 1.0000x reference)
import functools

import jax
import jax.numpy as jnp
from jax import lax
from jax.experimental import pallas as pl
from jax.experimental.pallas import tpu as pltpu
from jax.experimental.pallas import tpu_sc as plsc

_NC = 2
_NS = 16
_NW = _NC * _NS
_L = 16

_CHUNK = 128


def _matvec_body(tbl_ref, w_ref, b_ref, out_ref):
    prod = jax.lax.dot_general(
        w_ref[...], tbl_ref[...],
        (((1,), (1,)), ((), ())),
        preferred_element_type=jnp.float32,
    )
    out_ref[...] = prod.reshape(-1) + b_ref[0]


def _project_table(table, w_row, b_scaled, blk):
    vocab, embed = table.shape
    grid = (vocab + blk - 1) // blk
    return pl.pallas_call(
        _matvec_body,
        grid=(grid,),
        in_specs=[
            pl.BlockSpec((blk, embed), lambda i: (i, 0)),
            pl.BlockSpec((1, embed), lambda i: (0, 0)),
            pl.BlockSpec(memory_space=pltpu.SMEM),
        ],
        out_specs=pl.BlockSpec((blk,), lambda i: (i,)),
        out_shape=jax.ShapeDtypeStruct((vocab,), jnp.float32),
    )(table, w_row, b_scaled)


def _transpose_body(x_ref, out_ref):
    out_ref[...] = jnp.swapaxes(x_ref[...], 1, 2).reshape(-1)


def _transpose_indices(x4):
    nw, bpw, hist = x4.shape
    return pl.pallas_call(
        _transpose_body,
        grid=(nw,),
        in_specs=[pl.BlockSpec((1, bpw, hist), lambda i: (i, 0, 0))],
        out_specs=pl.BlockSpec((hist * bpw,), lambda i: (i,)),
        out_shape=jax.ShapeDtypeStruct((nw * hist * bpw,), jnp.int32),
    )(x4)


def _make_sc_gather(batch, hist, ipw, nch):
    bpw = batch // _NW
    mesh = plsc.VectorSubcoreMesh(
        core_axis_name="c", subcore_axis_name="s",
        num_cores=_NC, num_subcores=_NS,
    )

    @functools.partial(
        pl.kernel,
        mesh=mesh,
        out_type=jax.ShapeDtypeStruct((batch,), jnp.float32),
        scratch_types=[
            pltpu.VMEM((ipw,), jnp.int32),
            pltpu.VMEM((ipw,), jnp.float32),
            pltpu.VMEM((bpw,), jnp.float32),
            pltpu.SemaphoreType.DMA,
        ],
    )
    def sc_gather(p_hbm, x_hbm, out_hbm, idx_v, vals_v, res_v, sem):
        wid = lax.axis_index("s") * _NC + lax.axis_index("c")
        pltpu.sync_copy(x_hbm.at[pl.ds(wid * ipw, ipw)], idx_v)

        k = 8
        ngrp = nch // k

        def fire(g):
            for i in range(k):
                c = g * k + i
                pltpu.async_copy(
                    p_hbm.at[idx_v.at[pl.ds(c * _CHUNK, _CHUNK)]],
                    vals_v.at[pl.ds(c * _CHUNK, _CHUNK)],
                    sem,
                )

        def drain(g):
            pltpu.make_async_copy(
                p_hbm.at[pl.ds(0, k * _CHUNK)],
                vals_v.at[pl.ds(g * k * _CHUNK, k * _CHUNK)],
                sem,
            ).wait()

        fire(0)

        def ring(g, carry):
            fire(g + 1)
            drain(g)
            return carry

        lax.fori_loop(0, ngrp - 1, ring, 0)
        drain(ngrp - 1)

        def grp_body(g, carry):
            acc = jnp.zeros((_L,), jnp.float32)
            for j in range(hist):
                acc = acc + vals_v[pl.ds(j * bpw + g * _L, _L)]
            res_v[pl.ds(g * _L, _L)] = acc
            return carry

        lax.fori_loop(0, bpw // _L, grp_body, 0)
        pltpu.sync_copy(res_v, out_hbm.at[pl.ds(wid * bpw, bpw)])

    return sc_gather


def kernel(x, table, W, b):
    batch, hist = x.shape
    vocab, embed = table.shape

    inv_h = 1.0 / hist
    w_row = jnp.pad(
        (W * inv_h).reshape(1, embed), ((0, 0), (0, 128 - embed))
    )
    b_scaled = (b * inv_h).astype(jnp.float32)

    table_pad = jnp.pad(table, ((0, 0), (0, 128 - embed)))
    p = _project_table(table_pad, w_row, b_scaled, blk=32768)

    bpw = batch // _NW
    ipw = bpw * hist
    nch = ipw // _CHUNK
    x4 = x.astype(jnp.int32).reshape(_NW, bpw, hist)
    xt = _transpose_indices(x4)

    out = _make_sc_gather(batch, hist, ipw, nch)(p, xt)
    return out.reshape(batch, 1)


# speedup vs baseline: 1.4272x; 1.4272x over previous
import functools

import jax
import jax.numpy as jnp
from jax import lax
from jax.experimental import pallas as pl
from jax.experimental.pallas import tpu as pltpu
from jax.experimental.pallas import tpu_sc as plsc

_NC = 2
_NS = 16
_NW = _NC * _NS
_L = 16

_CHUNK = 128


def _matvec_body(tbl_ref, w_ref, b_ref, out_ref):
    prod = jax.lax.dot_general(
        w_ref[...], tbl_ref[...],
        (((1,), (1,)), ((), ())),
        preferred_element_type=jnp.float32,
    )
    out_ref[...] = prod.reshape(-1) + b_ref[0]


def _project_table(table, w_row, b_scaled, blk):
    vocab, embed = table.shape
    grid = (vocab + blk - 1) // blk
    return pl.pallas_call(
        _matvec_body,
        grid=(grid,),
        in_specs=[
            pl.BlockSpec((blk, embed), lambda i: (i, 0)),
            pl.BlockSpec((1, embed), lambda i: (0, 0)),
            pl.BlockSpec(memory_space=pltpu.SMEM),
        ],
        out_specs=pl.BlockSpec((blk,), lambda i: (i,)),
        out_shape=jax.ShapeDtypeStruct((vocab,), jnp.float32),
    )(table, w_row, b_scaled)


def _transpose_body(x_ref, out_ref):
    out_ref[...] = jnp.swapaxes(x_ref[...], 1, 2).reshape(-1)


def _transpose_indices(x4):
    nw, bpw, hist = x4.shape
    return pl.pallas_call(
        _transpose_body,
        grid=(nw,),
        in_specs=[pl.BlockSpec((1, bpw, hist), lambda i: (i, 0, 0))],
        out_specs=pl.BlockSpec((hist * bpw,), lambda i: (i,)),
        out_shape=jax.ShapeDtypeStruct((nw * hist * bpw,), jnp.int32),
    )(x4)


def _make_sc_gather(batch, hist, ipw, nch):
    bpw = batch // _NW
    mesh = plsc.VectorSubcoreMesh(
        core_axis_name="c", subcore_axis_name="s",
        num_cores=_NC, num_subcores=_NS,
    )

    @functools.partial(
        pl.kernel,
        mesh=mesh,
        out_type=jax.ShapeDtypeStruct((batch,), jnp.float32),
        scratch_types=[
            pltpu.VMEM((ipw,), jnp.int32),
            pltpu.VMEM((ipw,), jnp.float32),
            pltpu.VMEM((bpw,), jnp.float32),
            pltpu.SemaphoreType.DMA,
        ],
    )
    def sc_gather(p_hbm, x_hbm, out_hbm, idx_v, vals_v, res_v, sem):
        wid = lax.axis_index("s") * _NC + lax.axis_index("c")
        pltpu.sync_copy(x_hbm.at[pl.ds(wid * ipw, ipw)], idx_v)

        k = 8
        ngrp = nch // k

        def fire(g):
            for i in range(k):
                c = g * k + i
                pltpu.async_copy(
                    p_hbm.at[idx_v.at[pl.ds(c * _CHUNK, _CHUNK)]],
                    vals_v.at[pl.ds(c * _CHUNK, _CHUNK)],
                    sem,
                )

        def drain(g):
            pltpu.make_async_copy(
                p_hbm.at[pl.ds(0, k * _CHUNK)],
                vals_v.at[pl.ds(g * k * _CHUNK, k * _CHUNK)],
                sem,
            ).wait()

        fire(0)
        fire(1)

        def ring(g, carry):
            fire(g + 2)
            drain(g)
            return carry

        lax.fori_loop(0, ngrp - 2, ring, 0)
        drain(ngrp - 2)
        drain(ngrp - 1)

        def grp_body(g, carry):
            acc = jnp.zeros((_L,), jnp.float32)
            for j in range(hist):
                acc = acc + vals_v[pl.ds(j * bpw + g * _L, _L)]
            res_v[pl.ds(g * _L, _L)] = acc
            return carry

        lax.fori_loop(0, bpw // _L, grp_body, 0)
        pltpu.sync_copy(res_v, out_hbm.at[pl.ds(wid * bpw, bpw)])

    return sc_gather


def kernel(x, table, W, b):
    batch, hist = x.shape
    vocab, embed = table.shape

    inv_h = 1.0 / hist
    w_row = (W * inv_h).reshape(1, embed).astype(jnp.bfloat16)
    b_scaled = (b * inv_h).astype(jnp.float32)

    table_bf = table.astype(jnp.bfloat16)
    p = _project_table(table_bf, w_row, b_scaled, blk=65536)

    bpw = batch // _NW
    ipw = bpw * hist
    nch = ipw // _CHUNK
    x4 = x.astype(jnp.int32).reshape(_NW, bpw, hist)
    xt = _transpose_indices(x4)

    out = _make_sc_gather(batch, hist, ipw, nch)(p, xt)
    return out.reshape(batch, 1)


# speedup vs baseline: 1.4393x; 1.0084x over previous
import functools

import jax
import jax.numpy as jnp
from jax import lax
from jax.experimental import pallas as pl
from jax.experimental.pallas import tpu as pltpu
from jax.experimental.pallas import tpu_sc as plsc

_NC = 2
_NS = 16
_NW = _NC * _NS
_L = 16

_CHUNK = 512


def _matvec_body(tbl_ref, w_ref, b_ref, out_ref):
    prod = jax.lax.dot_general(
        w_ref[...], tbl_ref[...],
        (((1,), (1,)), ((), ())),
        preferred_element_type=jnp.float32,
    )
    out_ref[...] = prod.reshape(-1) + b_ref[0]


def _project_table(table, w_row, b_scaled, blk):
    vocab, embed = table.shape
    grid = (vocab + blk - 1) // blk
    return pl.pallas_call(
        _matvec_body,
        grid=(grid,),
        in_specs=[
            pl.BlockSpec((blk, embed), lambda i: (i, 0)),
            pl.BlockSpec((1, embed), lambda i: (0, 0)),
            pl.BlockSpec(memory_space=pltpu.SMEM),
        ],
        out_specs=pl.BlockSpec((blk,), lambda i: (i,)),
        out_shape=jax.ShapeDtypeStruct((vocab,), jnp.float32),
    )(table, w_row, b_scaled)


def _transpose_body(x_ref, out_ref):
    out_ref[...] = jnp.swapaxes(x_ref[...], 1, 2).reshape(-1)


def _transpose_indices(x4):
    nw, bpw, hist = x4.shape
    return pl.pallas_call(
        _transpose_body,
        grid=(nw,),
        in_specs=[pl.BlockSpec((1, bpw, hist), lambda i: (i, 0, 0))],
        out_specs=pl.BlockSpec((hist * bpw,), lambda i: (i,)),
        out_shape=jax.ShapeDtypeStruct((nw * hist * bpw,), jnp.int32),
    )(x4)


def _make_sc_gather(batch, hist, ipw, nch):
    bpw = batch // _NW
    mesh = plsc.VectorSubcoreMesh(
        core_axis_name="c", subcore_axis_name="s",
        num_cores=_NC, num_subcores=_NS,
    )

    @functools.partial(
        pl.kernel,
        mesh=mesh,
        out_type=jax.ShapeDtypeStruct((batch,), jnp.float32),
        scratch_types=[
            pltpu.VMEM((ipw,), jnp.int32),
            pltpu.VMEM((ipw,), jnp.float32),
            pltpu.VMEM((bpw,), jnp.float32),
            pltpu.SemaphoreType.DMA,
        ],
    )
    def sc_gather(p_hbm, x_hbm, out_hbm, idx_v, vals_v, res_v, sem):
        wid = lax.axis_index("s") * _NC + lax.axis_index("c")
        pltpu.sync_copy(x_hbm.at[pl.ds(wid * ipw, ipw)], idx_v)

        k = 5
        ngrp = nch // k

        def fire(g):
            for i in range(k):
                c = g * k + i
                pltpu.async_copy(
                    p_hbm.at[idx_v.at[pl.ds(c * _CHUNK, _CHUNK)]],
                    vals_v.at[pl.ds(c * _CHUNK, _CHUNK)],
                    sem,
                )

        def drain(g):
            pltpu.make_async_copy(
                p_hbm.at[pl.ds(0, k * _CHUNK)],
                vals_v.at[pl.ds(g * k * _CHUNK, k * _CHUNK)],
                sem,
            ).wait()

        fire(0)
        fire(1)

        def ring(g, carry):
            fire(g + 2)
            drain(g)
            return carry

        lax.fori_loop(0, ngrp - 2, ring, 0)
        drain(ngrp - 2)
        drain(ngrp - 1)

        def grp_body(g, carry):
            acc = jnp.zeros((_L,), jnp.float32)
            for j in range(hist):
                acc = acc + vals_v[pl.ds(j * bpw + g * _L, _L)]
            res_v[pl.ds(g * _L, _L)] = acc
            return carry

        lax.fori_loop(0, bpw // _L, grp_body, 0)
        pltpu.sync_copy(res_v, out_hbm.at[pl.ds(wid * bpw, bpw)])

    return sc_gather


def kernel(x, table, W, b):
    batch, hist = x.shape
    vocab, embed = table.shape

    inv_h = 1.0 / hist
    w_row = (W * inv_h).reshape(1, embed).astype(jnp.bfloat16)
    b_scaled = (b * inv_h).astype(jnp.float32)

    table_bf = table.astype(jnp.bfloat16)
    p = _project_table(table_bf, w_row, b_scaled, blk=65536)

    bpw = batch // _NW
    ipw = bpw * hist
    nch = ipw // _CHUNK
    x4 = x.astype(jnp.int32).reshape(_NW, bpw, hist)
    xt = _transpose_indices(x4)

    out = _make_sc_gather(batch, hist, ipw, nch)(p, xt)
    return out.reshape(batch, 1)
